# baseline (speedup 1.0000x reference)
import jax
import jax.numpy as jnp
from jax import lax
from jax.experimental import pallas as pl
from jax.experimental.pallas import tpu as pltpu
from jax.experimental.pallas import tpu_sc as plsc

VOCAB = 100000
EMBED = 128
PROJ = 128
B = 1024
L = 200

NC = 2
NS = 16
NW = NC * NS

TOKENS = B * L
PER_W = TOKENS // NW
CHUNK = 100
NCHUNK = PER_W // CHUNK
NBUF = 4
SETROWS = NBUF * CHUNK
NPAIR = NCHUNK // (2 * NBUF)

ROW_BLOCK = 25000


def _project_body(x_ref, wt_ref, b_ref, out_ref):
    out_ref[...] = (
        jnp.dot(x_ref[...], wt_ref[...], preferred_element_type=jnp.float32)
        + b_ref[...]
    )


def _project_table(table, Wt, bias_row):
    return pl.pallas_call(
        _project_body,
        grid=(VOCAB // ROW_BLOCK,),
        in_specs=[
            pl.BlockSpec((ROW_BLOCK, EMBED), lambda i: (i, 0)),
            pl.BlockSpec((EMBED, PROJ), lambda i: (0, 0)),
            pl.BlockSpec((1, PROJ), lambda i: (0, 0)),
        ],
        out_specs=pl.BlockSpec((ROW_BLOCK, PROJ), lambda i: (i, 0)),
        out_shape=jax.ShapeDtypeStruct((VOCAB, PROJ), jnp.float32),
    )(table, Wt, bias_row)


def _gather_kernel_body(ptab, ids, out, idx_v, bufA, bufB, gsemsA, gsemsB, osemA, osemB):
    wid = lax.axis_index("c") * NS + lax.axis_index("s")
    base = wid * PER_W
    pltpu.sync_copy(ids.at[wid], idx_v)

    @pl.loop(0, NPAIR)
    def _(p):
        def run_set(buf, gsems, osem, g):
            @pl.when(p > 0)
            def _():
                pltpu.make_async_copy(buf, out.at[pl.ds(base, SETROWS)], osem).wait()

            gh = []
            for bi in range(NBUF):
                gh.append(
                    pltpu.async_copy(
                        ptab.at[idx_v.at[g * NBUF + bi]],
                        buf.at[pl.ds(bi * CHUNK, CHUNK)],
                        gsems[bi],
                    )
                )
            return gh

        ghA = run_set(bufA, gsemsA, osemA, 2 * p)
        ghB = run_set(bufB, gsemsB, osemB, 2 * p + 1)
        for h in ghA:
            h.wait()
        pltpu.async_copy(bufA, out.at[pl.ds(base + 2 * p * SETROWS, SETROWS)], osemA)
        for h in ghB:
            h.wait()
        pltpu.async_copy(
            bufB, out.at[pl.ds(base + (2 * p + 1) * SETROWS, SETROWS)], osemB
        )

    pltpu.make_async_copy(bufA, out.at[pl.ds(base, SETROWS)], osemA).wait()
    pltpu.make_async_copy(bufB, out.at[pl.ds(base, SETROWS)], osemB).wait()


def _gather_rows(ptab, ids3d):
    mesh = plsc.VectorSubcoreMesh(
        core_axis_name="c", subcore_axis_name="s", num_cores=NC, num_subcores=NS
    )
    return pl.kernel(
        _gather_kernel_body,
        out_type=jax.ShapeDtypeStruct((TOKENS, PROJ), jnp.float32),
        mesh=mesh,
        scratch_types=[
            pltpu.VMEM((NCHUNK, CHUNK), jnp.int32),
            pltpu.VMEM((SETROWS, PROJ), jnp.float32),
            pltpu.VMEM((SETROWS, PROJ), jnp.float32),
            [pltpu.SemaphoreType.DMA for _ in range(NBUF)],
            [pltpu.SemaphoreType.DMA for _ in range(NBUF)],
            pltpu.SemaphoreType.DMA,
            pltpu.SemaphoreType.DMA,
        ],
    )(ptab, ids3d)


def kernel(ext_word_ids, seq_lengths, embed_table, W, b):
    del seq_lengths
    ids3d = ext_word_ids.astype(jnp.int32).reshape(NW, NCHUNK, CHUNK)
    ptab = _project_table(embed_table, W.T, b.reshape(1, PROJ))
    flat = _gather_rows(ptab, ids3d)
    return flat.reshape(B, L, PROJ)


# speedup vs baseline: 1.0180x; 1.0180x over previous
import jax
import jax.numpy as jnp
from jax import lax
from jax.experimental import pallas as pl
from jax.experimental.pallas import tpu as pltpu
from jax.experimental.pallas import tpu_sc as plsc

VOCAB = 100000
EMBED = 128
PROJ = 128
B = 1024
L = 200

NC = 2
NS = 16
NW = NC * NS

TOKENS = B * L
PER_W = TOKENS // NW
CHUNK = 64
NCHUNK = PER_W // CHUNK
NBUF = 5
SETROWS = NBUF * CHUNK
NPAIR = NCHUNK // (2 * NBUF)

ROW_BLOCK = 20000


def _project_body(x_ref, wt_ref, b_ref, out_ref):
    out_ref[...] = (
        jnp.dot(x_ref[...], wt_ref[...], preferred_element_type=jnp.float32)
        + b_ref[...]
    )


def _project_table(table, Wt, bias_row):
    return pl.pallas_call(
        _project_body,
        grid=(VOCAB // ROW_BLOCK,),
        in_specs=[
            pl.BlockSpec((ROW_BLOCK, EMBED), lambda i: (i, 0)),
            pl.BlockSpec((EMBED, PROJ), lambda i: (0, 0)),
            pl.BlockSpec((1, PROJ), lambda i: (0, 0)),
        ],
        out_specs=pl.BlockSpec((ROW_BLOCK, PROJ), lambda i: (i, 0)),
        out_shape=jax.ShapeDtypeStruct((VOCAB, PROJ), jnp.float32),
    )(table, Wt, bias_row)


def _gather_kernel_body(ptab, ids, out, idx_v, bufA, bufB, gsemsA, gsemsB, osemA, osemB):
    wid = lax.axis_index("s") * NC + lax.axis_index("c")
    base = wid * PER_W
    pltpu.sync_copy(ids.at[wid], idx_v)

    @pl.loop(0, NPAIR)
    def _(p):
        def run_set(buf, gsems, osem, g):
            @pl.when(p > 0)
            def _():
                pltpu.make_async_copy(buf, out.at[pl.ds(base, SETROWS)], osem).wait()

            gh = []
            for bi in range(NBUF):
                gh.append(
                    pltpu.async_copy(
                        ptab.at[idx_v.at[g * NBUF + bi]],
                        buf.at[pl.ds(bi * CHUNK, CHUNK)],
                        gsems[bi],
                    )
                )
            return gh

        ghA = run_set(bufA, gsemsA, osemA, 2 * p)
        ghB = run_set(bufB, gsemsB, osemB, 2 * p + 1)
        for h in ghA:
            h.wait()
        pltpu.async_copy(bufA, out.at[pl.ds(base + 2 * p * SETROWS, SETROWS)], osemA)
        for h in ghB:
            h.wait()
        pltpu.async_copy(
            bufB, out.at[pl.ds(base + (2 * p + 1) * SETROWS, SETROWS)], osemB
        )

    pltpu.make_async_copy(bufA, out.at[pl.ds(base, SETROWS)], osemA).wait()
    pltpu.make_async_copy(bufB, out.at[pl.ds(base, SETROWS)], osemB).wait()


def _gather_rows(ptab, ids3d):
    mesh = plsc.VectorSubcoreMesh(
        core_axis_name="c", subcore_axis_name="s", num_cores=NC, num_subcores=NS
    )
    return pl.kernel(
        _gather_kernel_body,
        out_type=jax.ShapeDtypeStruct((TOKENS, PROJ), jnp.float32),
        mesh=mesh,
        scratch_types=[
            pltpu.VMEM((NCHUNK, CHUNK), jnp.int32),
            pltpu.VMEM((SETROWS, PROJ), jnp.float32),
            pltpu.VMEM((SETROWS, PROJ), jnp.float32),
            [pltpu.SemaphoreType.DMA for _ in range(NBUF)],
            [pltpu.SemaphoreType.DMA for _ in range(NBUF)],
            pltpu.SemaphoreType.DMA,
            pltpu.SemaphoreType.DMA,
        ],
    )(ptab, ids3d)


def kernel(ext_word_ids, seq_lengths, embed_table, W, b):
    del seq_lengths
    ids3d = ext_word_ids.astype(jnp.int32).reshape(NW, NCHUNK, CHUNK)
    ptab = _project_table(embed_table, W.T, b.reshape(1, PROJ))
    flat = _gather_rows(ptab, ids3d)
    return flat.reshape(B, L, PROJ)


# speedup vs baseline: 1.0240x; 1.0059x over previous
import jax
import jax.numpy as jnp
from jax import lax
from jax.experimental import pallas as pl
from jax.experimental.pallas import tpu as pltpu
from jax.experimental.pallas import tpu_sc as plsc

VOCAB = 100000
EMBED = 128
PROJ = 128
B = 1024
L = 200

NC = 2
NS = 16
NW = NC * NS

TOKENS = B * L
PER_W = TOKENS // NW
CHUNK = 80
NCHUNK = PER_W // CHUNK
NBUF = 5
SETROWS = NBUF * CHUNK
NPAIR = NCHUNK // (2 * NBUF)

ROW_BLOCK = 20000


def _project_body(x_ref, wt_ref, b_ref, out_ref):
    out_ref[...] = (
        jnp.dot(x_ref[...], wt_ref[...], preferred_element_type=jnp.float32)
        + b_ref[...]
    )


def _project_table(table, Wt, bias_row):
    return pl.pallas_call(
        _project_body,
        grid=(VOCAB // ROW_BLOCK,),
        in_specs=[
            pl.BlockSpec((ROW_BLOCK, EMBED), lambda i: (i, 0)),
            pl.BlockSpec((EMBED, PROJ), lambda i: (0, 0)),
            pl.BlockSpec((1, PROJ), lambda i: (0, 0)),
        ],
        out_specs=pl.BlockSpec((ROW_BLOCK, PROJ), lambda i: (i, 0)),
        out_shape=jax.ShapeDtypeStruct((VOCAB, PROJ), jnp.float32),
    )(table, Wt, bias_row)


def _gather_kernel_body(ptab, ids, out, idx_v, bufA, bufB, gsemsA, gsemsB, osemA, osemB):
    wid = lax.axis_index("s") * NC + lax.axis_index("c")
    base = wid * PER_W
    pltpu.sync_copy(ids.at[wid], idx_v)

    @pl.loop(0, NPAIR)
    def _(p):
        def run_set(buf, gsems, osem, g):
            @pl.when(p > 0)
            def _():
                pltpu.make_async_copy(buf, out.at[pl.ds(base, SETROWS)], osem).wait()

            gh = []
            for bi in range(NBUF):
                gh.append(
                    pltpu.async_copy(
                        ptab.at[idx_v.at[g * NBUF + bi]],
                        buf.at[pl.ds(bi * CHUNK, CHUNK)],
                        gsems[bi],
                    )
                )
            return gh

        ghA = run_set(bufA, gsemsA, osemA, 2 * p)
        ghB = run_set(bufB, gsemsB, osemB, 2 * p + 1)
        for h in ghA:
            h.wait()
        pltpu.async_copy(bufA, out.at[pl.ds(base + 2 * p * SETROWS, SETROWS)], osemA)
        for h in ghB:
            h.wait()
        pltpu.async_copy(
            bufB, out.at[pl.ds(base + (2 * p + 1) * SETROWS, SETROWS)], osemB
        )

    pltpu.make_async_copy(bufA, out.at[pl.ds(base, SETROWS)], osemA).wait()
    pltpu.make_async_copy(bufB, out.at[pl.ds(base, SETROWS)], osemB).wait()


def _gather_rows(ptab, ids3d):
    mesh = plsc.VectorSubcoreMesh(
        core_axis_name="c", subcore_axis_name="s", num_cores=NC, num_subcores=NS
    )
    return pl.kernel(
        _gather_kernel_body,
        out_type=jax.ShapeDtypeStruct((TOKENS, PROJ), jnp.float32),
        mesh=mesh,
        scratch_types=[
            pltpu.VMEM((NCHUNK, CHUNK), jnp.int32),
            pltpu.VMEM((SETROWS, PROJ), jnp.float32),
            pltpu.VMEM((SETROWS, PROJ), jnp.float32),
            [pltpu.SemaphoreType.DMA for _ in range(NBUF)],
            [pltpu.SemaphoreType.DMA for _ in range(NBUF)],
            pltpu.SemaphoreType.DMA,
            pltpu.SemaphoreType.DMA,
        ],
    )(ptab, ids3d)


def kernel(ext_word_ids, seq_lengths, embed_table, W, b):
    del seq_lengths
    ids3d = ext_word_ids.astype(jnp.int32).reshape(NW, NCHUNK, CHUNK)
    ptab = _project_table(embed_table, W.T, b.reshape(1, PROJ))
    flat = _gather_rows(ptab, ids3d)
    return flat.reshape(B, L, PROJ)
